# baseline (speedup 1.0000x reference)
import jax
import jax.numpy as jnp
from jax.experimental import pallas as pl
from jax.experimental.pallas import tpu as pltpu

NUM_EXPERTS = 64
TOP_K = 8
HIDDEN = 1024
INTER = 512
BATCH = 128

_NEG = -3e38


def _moe_body(logits_ref, xT_ref, wg_ref, wu_ref, wd_ref, outT_ref, comb_ref):
    e = pl.program_id(0)

    @pl.when(e == 0)
    def _routing():
        lg = logits_ref[...]
        ids = jax.lax.broadcasted_iota(jnp.int32, (BATCH, NUM_EXPERTS), 1)
        work = lg
        vals = []
        sels = []
        for _ in range(TOP_K):
            m = jnp.max(work, axis=1, keepdims=True)
            is_m = work == m
            first = jnp.min(jnp.where(is_m, ids, NUM_EXPERTS), axis=1, keepdims=True)
            sel = ids == first
            vals.append(m)
            sels.append(sel)
            work = jnp.where(sel, _NEG, work)
        exps = [jnp.exp(v - vals[0]) for v in vals]
        denom = exps[0]
        for t in exps[1:]:
            denom = denom + t
        comb = jnp.zeros((BATCH, NUM_EXPERTS), jnp.float32)
        for k in range(TOP_K):
            comb = comb + sels[k].astype(jnp.float32) * (exps[k] / denom)
        comb_ref[...] = comb.T

    xT = xT_ref[...]
    wg = wg_ref[0]
    wu = wu_ref[0]
    wd = wd_ref[0]
    dn = (((1,), (0,)), ((), ()))
    g = jax.lax.dot_general(wg, xT, dn, preferred_element_type=jnp.float32,
                            precision=jax.lax.Precision.HIGHEST)
    u = jax.lax.dot_general(wu, xT, dn, preferred_element_type=jnp.float32,
                            precision=jax.lax.Precision.HIGHEST)
    h = (g * jax.nn.sigmoid(g)) * u
    y = jax.lax.dot_general(wd, h, dn, preferred_element_type=jnp.float32,
                            precision=jax.lax.Precision.HIGHEST)
    c = comb_ref[pl.ds(e, 1), :]

    @pl.when(e == 0)
    def _init():
        outT_ref[...] = y * c

    @pl.when(e > 0)
    def _acc():
        outT_ref[...] += y * c


def kernel(x, gate_w, w_gate, w_up, w_down):
    if x.ndim == 3:
        x2 = x[:, -1, :]
    else:
        x2 = x
    logits = (x2.astype(jnp.float16) @ gate_w.T.astype(jnp.float16)).astype(x2.dtype)
    xT = x2.T

    outT = pl.pallas_call(
        _moe_body,
        grid=(NUM_EXPERTS,),
        in_specs=[
            pl.BlockSpec((BATCH, NUM_EXPERTS), lambda e: (0, 0)),
            pl.BlockSpec((HIDDEN, BATCH), lambda e: (0, 0)),
            pl.BlockSpec((1, INTER, HIDDEN), lambda e: (e, 0, 0)),
            pl.BlockSpec((1, INTER, HIDDEN), lambda e: (e, 0, 0)),
            pl.BlockSpec((1, HIDDEN, INTER), lambda e: (e, 0, 0)),
        ],
        out_specs=pl.BlockSpec((HIDDEN, BATCH), lambda e: (0, 0)),
        out_shape=jax.ShapeDtypeStruct((HIDDEN, BATCH), jnp.float32),
        scratch_shapes=[pltpu.VMEM((NUM_EXPERTS, BATCH), jnp.float32)],
        compiler_params=pltpu.CompilerParams(
            dimension_semantics=("arbitrary",),
        ),
    )(logits, xT, w_gate, w_up, w_down)
    return outT.T


# speedup vs baseline: 2.4680x; 2.4680x over previous
import jax
import jax.numpy as jnp
from jax.experimental import pallas as pl
from jax.experimental.pallas import tpu as pltpu

NUM_EXPERTS = 64
TOP_K = 8
HIDDEN = 1024
INTER = 512
BATCH = 128

_NEG = -3e38


def _moe_body(logits_ref, xT_ref, wg_ref, wu_ref, wd_ref, outT_ref, comb_ref):
    e = pl.program_id(0)

    @pl.when(e == 0)
    def _routing():
        lg = logits_ref[...]
        ids = jax.lax.broadcasted_iota(jnp.int32, (BATCH, NUM_EXPERTS), 1)
        work = lg
        vals = []
        sels = []
        for _ in range(TOP_K):
            m = jnp.max(work, axis=1, keepdims=True)
            is_m = work == m
            first = jnp.min(jnp.where(is_m, ids, NUM_EXPERTS), axis=1, keepdims=True)
            sel = ids == first
            vals.append(m)
            sels.append(sel)
            work = jnp.where(sel, _NEG, work)
        exps = [jnp.exp(v - vals[0]) for v in vals]
        denom = exps[0]
        for t in exps[1:]:
            denom = denom + t
        comb = jnp.zeros((BATCH, NUM_EXPERTS), jnp.float32)
        for k in range(TOP_K):
            comb = comb + sels[k].astype(jnp.float32) * (exps[k] / denom)
        comb_ref[...] = comb.T

    xT = xT_ref[...]
    wg = wg_ref[0]
    wu = wu_ref[0]
    wd = wd_ref[0]
    dn = (((1,), (0,)), ((), ()))
    g = jax.lax.dot_general(wg, xT, dn, preferred_element_type=jnp.float32)
    u = jax.lax.dot_general(wu, xT, dn, preferred_element_type=jnp.float32)
    h = (g * jax.nn.sigmoid(g)) * u
    y = jax.lax.dot_general(wd, h, dn, preferred_element_type=jnp.float32)
    c = comb_ref[pl.ds(e, 1), :]

    @pl.when(e == 0)
    def _init():
        outT_ref[...] = y * c

    @pl.when(e > 0)
    def _acc():
        outT_ref[...] += y * c


def kernel(x, gate_w, w_gate, w_up, w_down):
    if x.ndim == 3:
        x2 = x[:, -1, :]
    else:
        x2 = x
    logits = (x2.astype(jnp.float16) @ gate_w.T.astype(jnp.float16)).astype(x2.dtype)
    xT = x2.T

    outT = pl.pallas_call(
        _moe_body,
        grid=(NUM_EXPERTS,),
        in_specs=[
            pl.BlockSpec((BATCH, NUM_EXPERTS), lambda e: (0, 0)),
            pl.BlockSpec((HIDDEN, BATCH), lambda e: (0, 0)),
            pl.BlockSpec((1, INTER, HIDDEN), lambda e: (e, 0, 0)),
            pl.BlockSpec((1, INTER, HIDDEN), lambda e: (e, 0, 0)),
            pl.BlockSpec((1, HIDDEN, INTER), lambda e: (e, 0, 0)),
        ],
        out_specs=pl.BlockSpec((HIDDEN, BATCH), lambda e: (0, 0)),
        out_shape=jax.ShapeDtypeStruct((HIDDEN, BATCH), jnp.float32),
        scratch_shapes=[pltpu.VMEM((NUM_EXPERTS, BATCH), jnp.float32)],
        compiler_params=pltpu.CompilerParams(
            dimension_semantics=("arbitrary",),
        ),
    )(logits, xT, w_gate, w_up, w_down)
    return outT.T
